# baseline (speedup 1.0000x reference)
import functools

import jax
import jax.numpy as jnp
from jax import lax
from jax.experimental import pallas as pl
from jax.experimental.pallas import tpu as pltpu
from jax.experimental.pallas import tpu_sc as plsc

VOCAB = 100000
DIM = 64
CHUNK = 128
G = 5


def _make_gather(n_total: int):
  info = plsc.get_sparse_core_info()
  nc, ns = info.num_cores, info.num_subcores
  nw = nc * ns
  assert n_total % (nw * CHUNK) == 0
  steps = n_total // (nw * CHUNK)
  b_per_w = steps * CHUNK

  mesh = plsc.VectorSubcoreMesh(core_axis_name="c", subcore_axis_name="s")

  assert steps % G == 0
  ng = steps // G
  assert ng % 2 == 0

  @functools.partial(
      pl.kernel,
      mesh=mesh,
      compiler_params=pltpu.CompilerParams(use_tc_tiling_on_sc=False),
      out_type=jax.ShapeDtypeStruct((n_total, DIM), jnp.float32),
      scratch_types=[
          pltpu.VMEM((steps, CHUNK), jnp.int32),
          pltpu.VMEM((2, G * CHUNK, DIM), jnp.float32),
          pltpu.SemaphoreType.DMA,
          pltpu.SemaphoreType.DMA,
      ],
  )
  def gather_kernel(idx_hbm, table_hbm, out_hbm, idx_v, rows_v, sem0, sem1):
    wid = lax.axis_index("s") * nc + lax.axis_index("c")
    out_base = wid * b_per_w
    sems = (sem0, sem1)
    pltpu.sync_copy(idx_hbm.at[wid], idx_v)

    def start_group(g, h):
      for j in range(G):
        pltpu.async_copy(
            table_hbm.at[idx_v.at[g * G + j]],
            rows_v.at[h].at[pl.ds(j * CHUNK, CHUNK)],
            sems[h],
        )

    def finish_group(g, h):
      for j in range(G):
        pltpu.make_async_copy(
            table_hbm.at[pl.ds(0, CHUNK)],
            rows_v.at[h].at[pl.ds(j * CHUNK, CHUNK)],
            sems[h],
        ).wait()
      pltpu.sync_copy(
          rows_v.at[h],
          out_hbm.at[pl.ds(out_base + g * (G * CHUNK), G * CHUNK)],
      )

    start_group(0, 0)

    def pair(gg, carry):
      g0 = 2 * gg
      start_group(g0 + 1, 1)
      finish_group(g0, 0)
      start_group(g0 + 2, 0)
      finish_group(g0 + 1, 1)
      return carry

    lax.fori_loop(0, ng // 2 - 1, pair, 0)
    start_group(ng - 1, 1)
    finish_group(ng - 2, 0)
    finish_group(ng - 1, 1)

  return gather_kernel


def kernel(x, table):
  b, s = x.shape
  n_total = b * s
  nw = 32
  idx3d = x.reshape(nw, n_total // (nw * CHUNK), CHUNK)
  out = _make_gather(n_total)(idx3d, table)
  return out.reshape(b, s, DIM)
